# Initial kernel scaffold; baseline (speedup 1.0000x reference)
import jax
import jax.numpy as jnp
from jax.experimental import pallas as pl


def kernel(x_user, x_recipient, x_device, x_ip, ei_transfer, ei_rev_transfer, ei_uses_device, ei_rev_uses_device, ei_uses_ip, ei_rev_uses_ip, Wp_user, bp_user, Wp_recipient, bp_recipient, Wp_device, bp_device, Wp_ip, bp_ip, Wl1, bl1, Wr1, gamma1, beta1, Wl2, bl2, Wr2, gamma2, beta2):
    raise NotImplementedError("write your pallas kernel here")


# speedup vs baseline: 1.7720x; 1.7720x over previous
import functools

import jax
import jax.numpy as jnp
from jax import lax
from jax.experimental import pallas as pl
from jax.experimental.pallas import tpu as pltpu
from jax.experimental.pallas import tpu_sc as plsc

EPS = 1e-5
NCORE = 2
NSUB = 16
BLK = 1024
CW = 16


def _ru(x, m):
    return ((x + m - 1) // m) * m


def _acc_rows(n_dst):
    return _ru(n_dst // 4 + 257, 512)


def _cnt_rows(n_dst):
    return _ru(n_dst + 1025, 1024)


@functools.lru_cache(maxsize=None)
def _sums_kernel(n_src, n_dst, e_pad):
    RA = _acc_rows(n_dst)
    rps = RA // NSUB
    e_sub = e_pad // NSUB
    n_blks = e_sub // BLK
    es128 = e_sub // 128
    mesh = plsc.VectorSubcoreMesh(core_axis_name="c", subcore_axis_name="s")

    def body(tab, gidx, sidx, z, out, gix_v, six_v, rows, zb, acc, sem):
        c = lax.axis_index("c")
        s = lax.axis_index("s")
        pltpu.sync_copy(z, zb)

        def zloop(i, carry):
            pltpu.sync_copy(
                zb, acc.at[pl.ds(pl.multiple_of(s * rps + i * 32, 32), 32)])
            return carry

        lax.fori_loop(0, rps // 32, zloop, 0)
        plsc.subcore_barrier()

        def blk_body(b, carry):
            def jloop(j, carry2):
                base = pl.multiple_of(s * e_sub + b * BLK + j * 128, 128)
                gbase = pl.multiple_of(c * e_pad + s * e_sub + b * BLK
                                       + j * 128, 128)
                pltpu.sync_copy(gidx.at[pl.ds(gbase, 128)], gix_v)
                pltpu.sync_copy(sidx.at[pl.ds(base, 128)], six_v)
                pltpu.async_copy(tab.at[gix_v], rows, sem).wait()
                pltpu.sync_copy(rows, acc.at[six_v], add=True)
                return carry2

            lax.fori_loop(0, BLK // 128, jloop, 0)
            return carry

        lax.fori_loop(0, n_blks, blk_body, 0)
        plsc.subcore_barrier()
        so = pl.multiple_of(s * rps, 32)
        oo = pl.multiple_of(c * RA + s * rps, 32)
        pltpu.sync_copy(acc.at[pl.ds(so, rps)], out.at[pl.ds(oo, rps)])

    return pl.kernel(
        body,
        out_type=jax.ShapeDtypeStruct((2 * RA, 128), jnp.float32),
        mesh=mesh,
        scratch_types=[
            pltpu.VMEM((128,), jnp.int32),
            pltpu.VMEM((128,), jnp.int32),
            pltpu.VMEM((128, 128), jnp.float32),
            pltpu.VMEM((32, 128), jnp.float32),
            pltpu.VMEM_SHARED((RA, 128), jnp.float32),
            pltpu.SemaphoreType.DMA,
        ],
    )


_BN = 2000


@functools.lru_cache(maxsize=None)
def _proj_kernel(n, d, h):
    def body(x_ref, w_ref, b_ref, o_ref):
        o_ref[...] = jax.lax.dot_general(
            x_ref[...], w_ref[...], (((1,), (1,)), ((), ())),
            preferred_element_type=jnp.float32) + b_ref[...]

    return pl.pallas_call(
        body,
        grid=(n // _BN,),
        in_specs=[
            pl.BlockSpec((_BN, d), lambda i: (i, 0)),
            pl.BlockSpec((h, d), lambda i: (0, 0)),
            pl.BlockSpec((1, h), lambda i: (0, 0)),
        ],
        out_specs=pl.BlockSpec((_BN, h), lambda i: (i, 0)),
        out_shape=jax.ShapeDtypeStruct((n, h), jnp.float32),
    )


@functools.lru_cache(maxsize=None)
def _fused_kernel(n, k, h):
    hh = h // 2

    def body(*refs):
        S0 = refs[0:k]
        S1 = refs[k:2 * k]
        C = refs[2 * k:3 * k]
        hin = refs[3 * k]
        A = refs[3 * k + 1:4 * k + 1]
        Bm = refs[4 * k + 1]
        bias = refs[4 * k + 2]
        g = refs[4 * k + 3]
        b2 = refs[4 * k + 4]
        o = refs[4 * k + 5]
        acc = jax.lax.dot_general(hin[...], Bm[...], (((1,), (0,)), ((), ())),
                                  preferred_element_type=jnp.float32)
        for j in range(k):
            inv = 1.0 / jnp.maximum(C[j][...], 1.0)
            Sf = jnp.concatenate([S0[j][...] * inv, S1[j][...] * inv], axis=1)
            acc = acc + jax.lax.dot_general(
                Sf, A[j][...], (((1,), (0,)), ((), ())),
                preferred_element_type=jnp.float32)
        acc = (acc + bias[...]) * g[...] + b2[...]
        o[...] = jnp.maximum(acc, 0.0)

    in_specs = (
        [pl.BlockSpec((_BN, hh), lambda i: (i, 0)) for _ in range(2 * k)]
        + [pl.BlockSpec((_BN, 1), lambda i: (i, 0)) for _ in range(k)]
        + [pl.BlockSpec((_BN, h), lambda i: (i, 0))]
        + [pl.BlockSpec((h, h), lambda i: (0, 0)) for _ in range(k + 1)]
        + [pl.BlockSpec((1, h), lambda i: (0, 0)) for _ in range(3)]
    )
    return pl.pallas_call(
        body,
        grid=(n // _BN,),
        in_specs=in_specs,
        out_specs=pl.BlockSpec((_BN, h), lambda i: (i, 0)),
        out_shape=jax.ShapeDtypeStruct((n, h), jnp.float32),
    )


def _prep_indices(ei, n_src, n_dst):
    E = ei.shape[1]
    e_pad = _ru(E, NSUB * NCORE * BLK)
    npad = e_pad - E
    src = ei[0]
    dst = ei[1]
    ar = jnp.arange(npad, dtype=jnp.int32)
    gq = (dst % 4) * n_src + src
    gq = jnp.concatenate([gq, ar % n_src])
    gidx = jnp.stack([gq, gq + 4 * n_src]).reshape(-1)
    sidx = jnp.concatenate([dst // 4, n_dst // 4 + (ar % 256)])
    cidx = jnp.concatenate([dst, n_dst + (ar % 1024)])
    cidx = cidx.reshape(e_pad // 128, 128)
    return gidx, sidx, cidx, e_pad


def _build_table(h):
    parts = []
    for c in range(2):
        half = h[:, 32 * c:32 * (c + 1)]
        for q in range(4):
            parts.append(jnp.pad(half, ((0, 0), (32 * q, 96 - 32 * q))))
    return jnp.concatenate(parts, axis=0)


def kernel(x_user, x_recipient, x_device, x_ip, ei_transfer, ei_rev_transfer,
           ei_uses_device, ei_rev_uses_device, ei_uses_ip, ei_rev_uses_ip,
           Wp_user, bp_user, Wp_recipient, bp_recipient, Wp_device, bp_device,
           Wp_ip, bp_ip, Wl1, bl1, Wr1, gamma1, beta1, Wl2, bl2, Wr2, gamma2,
           beta2):
    f32 = jnp.float32
    n_u = x_user.shape[0]
    n_d = x_device.shape[0]
    n_i = x_ip.shape[0]
    n_r = x_recipient.shape[0]
    d = x_user.shape[1]
    hc = Wp_user.shape[0]

    z = jnp.zeros((32, 128), f32)

    convs = [
        (ei_transfer, n_u, n_r),
        (ei_rev_transfer, n_r, n_u),
        (ei_uses_device, n_u, n_d),
        (ei_rev_uses_device, n_d, n_u),
        (ei_uses_ip, n_u, n_i),
        (ei_rev_uses_ip, n_i, n_u),
    ]
    idx = [_prep_indices(ei, ns, nd) for (ei, ns, nd) in convs]

    ones_tabs = {n: _build_table(jnp.ones((n, hc), f32))
                 for n in {n_u, n_r, n_d, n_i}}
    cnt = []
    for j in range(6):
        ns, nd = convs[j][1], convs[j][2]
        gidx, sidx, _, e_pad = idx[j]
        co = _sums_kernel(ns, nd, e_pad)(ones_tabs[ns], gidx, sidx, z)
        cnt.append(co[: nd // 4].reshape(nd, 32)[:, 0:1])

    hu = _proj_kernel(n_u, d, hc)(x_user, Wp_user, bp_user.reshape(1, hc))
    hr = _proj_kernel(n_r, d, hc)(x_recipient, Wp_recipient,
                                  bp_recipient.reshape(1, hc))
    hd = _proj_kernel(n_d, d, hc)(x_device, Wp_device, bp_device.reshape(1, hc))
    hi = _proj_kernel(n_i, d, hc)(x_ip, Wp_ip, bp_ip.reshape(1, hc))

    for (Wl, bl, Wr, gamma, beta) in ((Wl1, bl1, Wr1, gamma1, beta1),
                                      (Wl2, bl2, Wr2, gamma2, beta2)):
        tabs = {"u": _build_table(hu), "r": _build_table(hr),
                "d": _build_table(hd), "i": _build_table(hi)}
        srcs = ["u", "r", "u", "d", "u", "i"]

        S0, S1 = [], []
        for j in range(6):
            ns, nd = convs[j][1], convs[j][2]
            gidx, sidx, _, e_pad = idx[j]
            RA = _acc_rows(nd)
            out = _sums_kernel(ns, nd, e_pad)(tabs[srcs[j]], gidx, sidx, z)
            S0.append(out[: nd // 4].reshape(nd, 32))
            S1.append(out[RA: RA + nd // 4].reshape(nd, 32))

        gs = gamma / jnp.sqrt(1.0 + EPS)

        hr_new = _fused_kernel(n_r, 1, hc)(
            S0[0], S1[0], cnt[0], hr, Wl[0].T, Wr[0].T, bl[0].reshape(1, hc),
            gs[1].reshape(1, hc), beta[1].reshape(1, hc))
        hd_new = _fused_kernel(n_d, 1, hc)(
            S0[2], S1[2], cnt[2], hd, Wl[2].T, Wr[2].T, bl[2].reshape(1, hc),
            gs[2].reshape(1, hc), beta[2].reshape(1, hc))
        hi_new = _fused_kernel(n_i, 1, hc)(
            S0[4], S1[4], cnt[4], hi, Wl[4].T, Wr[4].T, bl[4].reshape(1, hc),
            gs[3].reshape(1, hc), beta[3].reshape(1, hc))
        third = f32(1.0 / 3.0)
        hu_new = _fused_kernel(n_u, 3, hc)(
            S0[1], S0[3], S0[5], S1[1], S1[3], S1[5],
            cnt[1], cnt[3], cnt[5], hu,
            Wl[1].T * third, Wl[3].T * third, Wl[5].T * third,
            (Wr[1] + Wr[3] + Wr[5]).T * third,
            ((bl[1] + bl[3] + bl[5]) * third).reshape(1, hc),
            gs[0].reshape(1, hc), beta[0].reshape(1, hc))

        hu, hr, hd, hi = hu_new, hr_new, hd_new, hi_new

    return (hu, hr, hd, hi)
